# baseline (speedup 1.0000x reference)
import jax
import jax.numpy as jnp
from jax.experimental import pallas as pl
from jax.experimental.pallas import tpu as pltpu

_NCH = 16


def _tabgnn_body(x_hbm, w1_ref, b1_ref, w2_ref, b2_ref, o_hbm,
                 xbuf, obuf, in_sem, out_sem):
    B, C, F = xbuf.shape
    CB = B // _NCH
    w1 = w1_ref[...]
    b1 = b1_ref[...]
    w2 = w2_ref[...]
    b2 = b2_ref[...]

    def in_copy(i):
        return pltpu.make_async_copy(
            x_hbm.at[pl.ds(i * CB, CB)], xbuf.at[pl.ds(i * CB, CB)],
            in_sem.at[i])

    def out_copy(i):
        return pltpu.make_async_copy(
            obuf.at[pl.ds(i * CB, CB)], o_hbm.at[pl.ds(i * CB, CB)],
            out_sem.at[i])

    for i in range(_NCH):
        in_copy(i).start()
    for i in range(_NCH):
        in_copy(i).wait()
        xc = xbuf[pl.ds(i * CB, CB)]
        m = jnp.mean(xc, axis=1)
        h = jnp.dot(m, w1, preferred_element_type=jnp.float32)
        h = jnp.maximum(h + b1, 0.0)
        o = jnp.dot(h, w2, preferred_element_type=jnp.float32) + b2
        obuf[pl.ds(i * CB, CB)] = jnp.broadcast_to(o[:, None, :], (CB, C, F))
        out_copy(i).start()
    for i in range(_NCH):
        out_copy(i).wait()


@jax.jit
def kernel(x, W1, b1, W2, b2):
    B, C, F = x.shape
    HID = W1.shape[1]
    vmem = pl.BlockSpec(memory_space=pltpu.MemorySpace.VMEM)
    return pl.pallas_call(
        _tabgnn_body,
        in_specs=[pl.BlockSpec(memory_space=pl.ANY), vmem, vmem, vmem, vmem],
        out_specs=pl.BlockSpec(memory_space=pl.ANY),
        out_shape=jax.ShapeDtypeStruct((B, C, F), x.dtype),
        scratch_shapes=[
            pltpu.VMEM((B, C, F), jnp.float32),
            pltpu.VMEM((B, C, F), jnp.float32),
            pltpu.SemaphoreType.DMA((_NCH,)),
            pltpu.SemaphoreType.DMA((_NCH,)),
        ],
    )(x, W1, b1.reshape(1, HID), W2, b2.reshape(1, F))
